# baseline (speedup 1.0000x reference)
import functools

import jax
import jax.numpy as jnp
from jax import lax
from jax.experimental import pallas as pl
from jax.experimental.pallas import tpu as pltpu
from jax.experimental.pallas import tpu_sc as plsc

B = 16384
EMB = 32
HID = 32

_info = plsc.get_sparse_core_info()
_NC, _NS = _info.num_cores, _info.num_subcores
_NW = _NC * _NS
_BPW = B // _NW

_mesh = plsc.VectorSubcoreMesh(core_axis_name="c", subcore_axis_name="s")


@functools.partial(
    pl.kernel,
    mesh=_mesh,
    compiler_params=pltpu.CompilerParams(use_tc_tiling_on_sc=False),
    out_type=[
        jax.ShapeDtypeStruct((B, EMB), jnp.float32),
        jax.ShapeDtypeStruct((B, EMB), jnp.float32),
    ],
    scratch_types=[
        pltpu.VMEM((_BPW,), jnp.int32),
        pltpu.VMEM((_BPW,), jnp.int32),
        pltpu.VMEM((_BPW, EMB), jnp.float32),
        pltpu.VMEM((_BPW, EMB), jnp.float32),
        pltpu.SemaphoreType.DMA,
        pltpu.SemaphoreType.DMA,
    ],
)
def _sc_gather(u_id_hbm, i_id_hbm, u_tab_hbm, i_tab_hbm,
               out_u_hbm, out_i_hbm,
               uidx_v, iidx_v, urows_v, irows_v, sem_u, sem_i):
    wid = lax.axis_index("s") * _NC + lax.axis_index("c")
    base = wid * _BPW
    pltpu.sync_copy(u_id_hbm.at[pl.ds(base, _BPW)], uidx_v)
    pltpu.sync_copy(i_id_hbm.at[pl.ds(base, _BPW)], iidx_v)
    cu = pltpu.async_copy(u_tab_hbm.at[uidx_v], urows_v, sem_u)
    ci = pltpu.async_copy(i_tab_hbm.at[iidx_v], irows_v, sem_i)
    cu.wait()
    ci.wait()
    pltpu.sync_copy(urows_v, out_u_hbm.at[pl.ds(base, _BPW)])
    pltpu.sync_copy(irows_v, out_i_hbm.at[pl.ds(base, _BPW)])


_BLK = 2048


def _mlp_body(uf_ref, itf_ref, ua_ref, ia_ref, w1_ref, b1_ref, w2_ref, b2_ref,
              out_ref):
    w1 = w1_ref[...]
    acc = jnp.dot(uf_ref[...], w1[0:EMB, :], preferred_element_type=jnp.float32)
    acc += jnp.dot(itf_ref[...], w1[EMB:2 * EMB, :],
                   preferred_element_type=jnp.float32)
    acc += jnp.dot(ua_ref[...], w1[2 * EMB:3 * EMB, :],
                   preferred_element_type=jnp.float32)
    acc += jnp.dot(ia_ref[...], w1[3 * EMB:4 * EMB, :],
                   preferred_element_type=jnp.float32)
    h = jnp.maximum(acc + b1_ref[...], 0.0)
    o = jnp.dot(h, w2_ref[...], preferred_element_type=jnp.float32)
    out_ref[...] = o + b2_ref[0, 0]


def _mlp(uf, itf, ua, ia, w1, b1, w2, b2):
    grid = (B // _BLK,)
    return pl.pallas_call(
        _mlp_body,
        grid=grid,
        in_specs=[
            pl.BlockSpec((_BLK, EMB), lambda i: (i, 0)),
            pl.BlockSpec((_BLK, EMB), lambda i: (i, 0)),
            pl.BlockSpec((_BLK, EMB), lambda i: (i, 0)),
            pl.BlockSpec((_BLK, EMB), lambda i: (i, 0)),
            pl.BlockSpec((4 * EMB, HID), lambda i: (0, 0)),
            pl.BlockSpec((1, HID), lambda i: (0, 0)),
            pl.BlockSpec((HID, 1), lambda i: (0, 0)),
            pl.BlockSpec((1, 1), lambda i: (0, 0)),
        ],
        out_specs=pl.BlockSpec((_BLK, 1), lambda i: (i, 0)),
        out_shape=jax.ShapeDtypeStruct((B, 1), jnp.float32),
    )(uf, itf, ua, ia, w1, b1, w2, b2)


def kernel(u_id, i_id, u_attr, i_attr, user_table, item_table, W1, b1, W2, b2):
    uf, itf = _sc_gather(u_id.astype(jnp.int32), i_id.astype(jnp.int32),
                         user_table, item_table)
    ua = jnp.squeeze(u_attr, axis=1)
    ia = jnp.squeeze(i_attr, axis=1)
    out = _mlp(uf, itf, ua, ia, W1, b1.reshape(1, HID), W2, b2.reshape(1, 1))
    return jnp.squeeze(out, axis=1)
